# baseline (speedup 1.0000x reference)
import functools

import jax
import jax.numpy as jnp
from jax import lax
from jax.experimental import pallas as pl
from jax.experimental.pallas import tpu as pltpu
from jax.experimental.pallas import tpu_sc as plsc

EMB = 128
NC = 2
NS = 16
NW = NC * NS
NBUF = 6
LOOK = 4


def _emb_body(n_b, seq, table_hbm, idx_hbm, out_hbm, idx_v, rows_v, *sems):
    gsems = sems[:NBUF]
    ssems = sems[NBUF:]
    wid = lax.axis_index("s") * NC + lax.axis_index("c")
    base = wid * n_b
    pltpu.sync_copy(idx_hbm.at[:, pl.ds(base, n_b)], idx_v)

    def fire_gather(j, b):
        pltpu.async_copy(table_hbm.at[idx_v.at[j]], rows_v.at[b], gsems[b])

    def wait_gather(j, b):
        pltpu.make_async_copy(
            table_hbm.at[idx_v.at[j]], rows_v.at[b], gsems[b]).wait()

    def out_slice(j):
        return out_hbm.at[j, pl.ds(base, n_b)]

    def fire_store(j, b):
        pltpu.async_copy(rows_v.at[b], out_slice(j), ssems[b])

    def wait_store(j, b):
        pltpu.make_async_copy(rows_v.at[b], out_slice(j), ssems[b]).wait()

    def body(j, carry):
        pltpu.async_copy(table_hbm.at[idx_v.at[j]], rows_v.at[0], gsems[0]).wait()
        pltpu.sync_copy(rows_v.at[0], out_slice(j))
        return carry

    lax.fori_loop(0, seq, body, 0)


@functools.partial(jax.jit, static_argnames=("n_b", "seq"))
def _emb_call(table, idx_t, n_b, seq):
    fn = pl.kernel(
        functools.partial(_emb_body, n_b, seq),
        mesh=plsc.VectorSubcoreMesh(core_axis_name="c", subcore_axis_name="s"),
        compiler_params=pltpu.CompilerParams(use_tc_tiling_on_sc=True),
        out_type=jax.ShapeDtypeStruct((seq, NW * n_b, EMB), jnp.float32),
        scratch_types=[
            pltpu.VMEM((seq, n_b), jnp.int32),
            pltpu.VMEM((NBUF, n_b, EMB), jnp.float32),
        ] + [pltpu.SemaphoreType.DMA] * (2 * NBUF),
    )
    return fn(table, idx_t)


def kernel(indices, table):
    bsz, seq = indices.shape
    assert bsz % NW == 0
    n_b = bsz // NW
    idx_t = indices.astype(jnp.int32).T
    out_t = _emb_call(table, idx_t, n_b, seq)
    return jnp.transpose(out_t, (1, 0, 2))


# speedup vs baseline: 1.4132x; 1.4132x over previous
import functools

import jax
import jax.numpy as jnp
from jax import lax
from jax.experimental import pallas as pl
from jax.experimental.pallas import tpu as pltpu
from jax.experimental.pallas import tpu_sc as plsc

EMB = 128
NC = 2
NS = 16
NW = NC * NS
NBUF = 3
LOOK = 2


def _emb_body(n_b, seq, table_hbm, idx_hbm, out_hbm, idx_v, rows_v, *sems):
    gsems = sems[:NBUF]
    ssems = sems[NBUF:]
    wid = lax.axis_index("s") * NC + lax.axis_index("c")
    base = wid * n_b
    pltpu.sync_copy(idx_hbm.at[:, pl.ds(base, n_b)], idx_v)
    T = seq // 2

    def fire_gather(t, b):
        for h in range(2):
            pltpu.async_copy(
                table_hbm.at[idx_v.at[2 * t + h]], rows_v.at[b, h], gsems[b])

    def wait_gather(t, b):
        for h in range(2):
            pltpu.make_async_copy(
                table_hbm.at[idx_v.at[2 * t + h]], rows_v.at[b, h],
                gsems[b]).wait()

    def out_slice(t):
        return out_hbm.at[pl.ds(2 * t, 2), pl.ds(base, n_b)]

    def fire_store(t, b):
        pltpu.async_copy(rows_v.at[b], out_slice(t), ssems[b])

    def wait_store(t, b):
        pltpu.make_async_copy(rows_v.at[b], out_slice(t), ssems[b]).wait()

    j0 = NBUF - LOOK
    assert T > NBUF
    for j in range(LOOK):
        fire_gather(j, j % NBUF)
    for j in range(j0):
        fire_gather(j + LOOK, (j + LOOK) % NBUF)
        wait_gather(j, j % NBUF)
        fire_store(j, j % NBUF)

    mid = T - NBUF
    grps = mid // NBUF

    def step(j, b, bf):
        wait_store(j - j0, bf)
        fire_gather(j + LOOK, bf)
        wait_gather(j, b)
        fire_store(j, b)

    def outer(g, carry):
        jg = j0 + g * NBUF
        for r in range(NBUF):
            step(jg + r, (j0 + r) % NBUF, (j0 + r + LOOK) % NBUF)
        return carry

    lax.fori_loop(0, grps, outer, 0)
    for j in range(j0 + grps * NBUF, T - LOOK):
        step(j, j % NBUF, (j + LOOK) % NBUF)

    for j in range(T - LOOK, T):
        wait_store(j - j0, (j + LOOK) % NBUF)
        wait_gather(j, j % NBUF)
        fire_store(j, j % NBUF)
    for j in range(T - j0, T):
        wait_store(j, j % NBUF)


@functools.partial(jax.jit, static_argnames=("n_b", "seq"))
def _emb_call(table, idx_t, n_b, seq):
    fn = pl.kernel(
        functools.partial(_emb_body, n_b, seq),
        mesh=plsc.VectorSubcoreMesh(core_axis_name="c", subcore_axis_name="s"),
        compiler_params=pltpu.CompilerParams(use_tc_tiling_on_sc=True),
        out_type=jax.ShapeDtypeStruct((seq, NW * n_b, EMB), jnp.float32),
        scratch_types=[
            pltpu.VMEM((seq, n_b), jnp.int32),
            pltpu.VMEM((NBUF, 2, n_b, EMB), jnp.float32),
        ] + [pltpu.SemaphoreType.DMA] * (2 * NBUF),
    )
    return fn(table, idx_t)


def kernel(indices, table):
    bsz, seq = indices.shape
    assert bsz % NW == 0 and seq % 2 == 0
    n_b = bsz // NW
    idx_t = indices.astype(jnp.int32).T
    out_t = _emb_call(table, idx_t, n_b, seq)
    return jnp.transpose(out_t, (1, 0, 2))


# speedup vs baseline: 1.4281x; 1.0106x over previous
import functools

import jax
import jax.numpy as jnp
from jax import lax
from jax.experimental import pallas as pl
from jax.experimental.pallas import tpu as pltpu
from jax.experimental.pallas import tpu_sc as plsc

EMB = 128
NC = 2
NS = 16
NW = NC * NS
NBUF = 6
LOOK = 4


def _emb_body(n_b, seq, table_hbm, idx_hbm, out_hbm, idx_v, rows_v, *sems):
    gsems = sems[:NBUF]
    ssems = sems[NBUF:]
    wid = lax.axis_index("s") * NC + lax.axis_index("c")
    base = wid * n_b
    pltpu.sync_copy(idx_hbm.at[:, pl.ds(base, n_b)], idx_v)

    def fire_gather(j, b):
        pltpu.async_copy(table_hbm.at[idx_v.at[j]], rows_v.at[b], gsems[b])

    def wait_gather(j, b):
        pltpu.make_async_copy(
            table_hbm.at[idx_v.at[j]], rows_v.at[b], gsems[b]).wait()

    def out_slice(j):
        return out_hbm.at[j, pl.ds(base, n_b)]

    def fire_store(j, b):
        pltpu.async_copy(rows_v.at[b], out_slice(j), ssems[b])

    def wait_store(j, b):
        pltpu.make_async_copy(rows_v.at[b], out_slice(j), ssems[b]).wait()

    j0 = NBUF - LOOK
    assert seq > NBUF
    for j in range(LOOK):
        fire_gather(j, j % NBUF)
    for j in range(j0):
        fire_gather(j + LOOK, (j + LOOK) % NBUF)
        wait_gather(j, j % NBUF)
        fire_store(j, j % NBUF)

    mid = seq - NBUF
    grps = mid // NBUF

    def step(j, b, bf):
        wait_store(j - j0, bf)
        fire_gather(j + LOOK, bf)
        wait_gather(j, b)
        fire_store(j, b)

    def outer(g, carry):
        jg = j0 + g * NBUF
        for r in range(NBUF):
            step(jg + r, (j0 + r) % NBUF, (j0 + r + LOOK) % NBUF)
        return carry

    lax.fori_loop(0, grps, outer, 0)
    for j in range(j0 + grps * NBUF, seq - LOOK):
        step(j, j % NBUF, (j + LOOK) % NBUF)

    for j in range(seq - LOOK, seq):
        wait_store(j - j0, (j + LOOK) % NBUF)
        wait_gather(j, j % NBUF)
        fire_store(j, j % NBUF)
    for j in range(seq - j0, seq):
        wait_store(j, j % NBUF)


@functools.partial(jax.jit, static_argnames=("n_b", "seq"))
def _emb_call(table, idx_t, n_b, seq):
    fn = pl.kernel(
        functools.partial(_emb_body, n_b, seq),
        mesh=plsc.VectorSubcoreMesh(core_axis_name="c", subcore_axis_name="s"),
        compiler_params=pltpu.CompilerParams(use_tc_tiling_on_sc=True),
        out_type=jax.ShapeDtypeStruct((seq, NW * n_b, EMB), jnp.float32),
        scratch_types=[
            pltpu.VMEM((seq, n_b), jnp.int32),
            pltpu.VMEM((NBUF, n_b, EMB), jnp.float32),
        ] + [pltpu.SemaphoreType.DMA] * (2 * NBUF),
    )
    return fn(table, idx_t)


def kernel(indices, table):
    bsz, seq = indices.shape
    assert bsz % NW == 0
    n_b = bsz // NW
    idx_t = indices.astype(jnp.int32).T
    out_t = _emb_call(table, idx_t, n_b, seq)
    return jnp.transpose(out_t, (1, 0, 2))
